# baseline (speedup 1.0000x reference)
import functools

import jax
import jax.numpy as jnp
from jax import lax
from jax.experimental import pallas as pl
from jax.experimental.pallas import tpu as pltpu
from jax.experimental.pallas import tpu_sc as plsc

_VOCAB = 1000
_NW = 32
_CHUNK = 64


def _head_table_body(e_ref, w_ref, p_ref):
    p_ref[...] = lax.dot_general(
        e_ref[...], w_ref[...],
        dimension_numbers=(((1,), (1,)), ((), ())),
        preferred_element_type=jnp.float32)


def _head_table(embed_table, head_w_padded):
    v = embed_table.shape[0]
    vp = head_w_padded.shape[0]
    return pl.pallas_call(
        _head_table_body,
        out_shape=jax.ShapeDtypeStruct((v, vp), jnp.float32),
    )(embed_table, head_w_padded)


@functools.partial(jax.jit, static_argnums=(2,))
def _gather_rows(ids, p, n_tokens):
    per_w = n_tokens // _NW
    n_chunks = per_w // _CHUNK
    vp = p.shape[1]
    mesh = plsc.VectorSubcoreMesh(core_axis_name="c", subcore_axis_name="s")

    @functools.partial(
        pl.kernel,
        out_type=jax.ShapeDtypeStruct((n_tokens, _VOCAB), jnp.float32),
        mesh=mesh,
        compiler_params=pltpu.CompilerParams(use_tc_tiling_on_sc=False),
        scratch_types=[
            pltpu.VMEM((per_w,), jnp.int32),
            pltpu.VMEM((_CHUNK, vp), jnp.float32),
            pltpu.VMEM((_CHUNK, vp), jnp.float32),
            pltpu.SemaphoreType.DMA,
            pltpu.SemaphoreType.DMA,
            pltpu.SemaphoreType.DMA,
            pltpu.SemaphoreType.DMA,
        ],
    )
    def gather(ids_hbm, p_hbm, out_hbm, idx_v, rows0, rows1,
               sg0, sg1, sw0, sw1):
        wid = lax.axis_index("s") * 2 + lax.axis_index("c")
        base = wid * per_w
        pltpu.sync_copy(ids_hbm.at[pl.ds(base, per_w)], idx_v)
        bufs = (rows0, rows1)
        gsems = (sg0, sg1)
        wsems = (sw0, sw1)

        def start_gather(c, b):
            off = pl.multiple_of(c * _CHUNK, _CHUNK)
            pltpu.async_copy(
                p_hbm.at[idx_v.at[pl.ds(off, _CHUNK)]], bufs[b], gsems[b])

        def start_write(c, b):
            off = pl.multiple_of(c * _CHUNK, _CHUNK)
            pltpu.async_copy(
                bufs[b], out_hbm.at[pl.ds(base + off, _CHUNK)], wsems[b])

        def wait_gather(b):
            pltpu.make_async_copy(p_hbm.at[idx_v.at[pl.ds(0, _CHUNK)]],
                                  bufs[b], gsems[b]).wait()

        def wait_write(b):
            pltpu.make_async_copy(bufs[b], out_hbm.at[pl.ds(0, _CHUNK)],
                                  wsems[b]).wait()

        start_gather(0, 0)
        start_gather(1, 1)
        n_pairs = n_chunks // 2

        def pair(j, carry):
            c = j * 2
            for b in range(2):
                wait_gather(b)
                start_write(c + b, b)
            for b in range(2):
                nxt = c + 2 + b
                @pl.when(nxt < n_chunks)
                def _():
                    wait_write(b)
                    start_gather(nxt, b)
            return carry

        lax.fori_loop(0, n_pairs, pair, 0)
        if n_chunks % 2:
            wait_gather(0)
            start_write(n_chunks - 1, 0)
        wait_write(0)
        wait_write(1)

    return gather(ids, p)


def kernel(input_ids, embed_table, head_w):
    b, t = input_ids.shape
    p = _head_table(embed_table, head_w)
    ids = input_ids.reshape(-1).astype(jnp.int32)
    out = _gather_rows(ids, p, b * t)
    return out.reshape(b, t, _VOCAB)


# speedup vs baseline: 1.4733x; 1.4733x over previous
import functools

import jax
import jax.numpy as jnp
from jax import lax
from jax.experimental import pallas as pl
from jax.experimental.pallas import tpu as pltpu

_BLK = 4


def _body(ids_ref, e_ref, w_ref, out_ref):
    n, _ = ids_ref.shape
    blk, t, v = out_ref.shape
    onehot = (ids_ref[...] == lax.broadcasted_iota(jnp.int32, (n, v), 1)
              ).astype(jnp.float32)
    x = lax.dot_general(onehot, e_ref[...],
                        dimension_numbers=(((1,), (0,)), ((), ())),
                        preferred_element_type=jnp.float32)
    y = lax.dot_general(x, w_ref[...],
                        dimension_numbers=(((1,), (1,)), ((), ())),
                        preferred_element_type=jnp.float32)
    out_ref[...] = y.reshape(blk, t, v)


def kernel(input_ids, embed_table, head_w):
    b, t = input_ids.shape
    v, d = embed_table.shape
    ids = input_ids.astype(jnp.int32).reshape(b * t, 1)
    grid = (b // _BLK,)
    return pl.pallas_call(
        _body,
        grid=grid,
        in_specs=[
            pl.BlockSpec((_BLK * t, 1), lambda i: (i, 0)),
            pl.BlockSpec((v, d), lambda i: (0, 0)),
            pl.BlockSpec((v, d), lambda i: (0, 0)),
        ],
        out_specs=pl.BlockSpec((_BLK, t, v), lambda i: (i, 0, 0)),
        out_shape=jax.ShapeDtypeStruct((b, t, v), jnp.float32),
    )(ids, embed_table, head_w)


# speedup vs baseline: 1.5208x; 1.0322x over previous
import functools

import jax
import jax.numpy as jnp
from jax import lax
from jax.experimental import pallas as pl
from jax.experimental.pallas import tpu as pltpu

_BLK = 4


def _body(ids_ref, e_ref, w_ref, out_ref):
    n, _ = ids_ref.shape
    blk, t, v = out_ref.shape
    onehot = (ids_ref[...] == lax.broadcasted_iota(jnp.int32, (n, v), 1)
              ).astype(jnp.bfloat16)
    x = lax.dot_general(onehot, e_ref[...],
                        dimension_numbers=(((1,), (0,)), ((), ())),
                        preferred_element_type=jnp.float32
                        ).astype(jnp.bfloat16)
    y = lax.dot_general(x, w_ref[...],
                        dimension_numbers=(((1,), (1,)), ((), ())),
                        preferred_element_type=jnp.float32)
    out_ref[...] = y.reshape(blk, t, v)


def kernel(input_ids, embed_table, head_w):
    b, t = input_ids.shape
    v, d = embed_table.shape
    ids = input_ids.astype(jnp.int32).reshape(b * t, 1)
    embed_table = embed_table.astype(jnp.bfloat16)
    head_w = head_w.astype(jnp.bfloat16)
    grid = (b // _BLK,)
    return pl.pallas_call(
        _body,
        grid=grid,
        in_specs=[
            pl.BlockSpec((_BLK * t, 1), lambda i: (i, 0)),
            pl.BlockSpec((v, d), lambda i: (0, 0)),
            pl.BlockSpec((v, d), lambda i: (0, 0)),
        ],
        out_specs=pl.BlockSpec((_BLK, t, v), lambda i: (i, 0, 0)),
        out_shape=jax.ShapeDtypeStruct((b, t, v), jnp.float32),
    )(ids, embed_table, head_w)


# speedup vs baseline: 1.7772x; 1.1686x over previous
import functools

import jax
import jax.numpy as jnp
from jax import lax
from jax.experimental import pallas as pl
from jax.experimental.pallas import tpu as pltpu

_BLK = 8


def _body(ids_ref, e_ref, w_ref, out_ref):
    n, _ = ids_ref.shape
    blk, t, v = out_ref.shape
    onehot = (ids_ref[...] == lax.broadcasted_iota(jnp.int32, (n, v), 1)
              ).astype(jnp.bfloat16)
    x = lax.dot_general(onehot, e_ref[...],
                        dimension_numbers=(((1,), (0,)), ((), ())),
                        preferred_element_type=jnp.float32
                        ).astype(jnp.bfloat16)
    y = lax.dot_general(x, w_ref[...],
                        dimension_numbers=(((1,), (1,)), ((), ())),
                        preferred_element_type=jnp.float32)
    out_ref[...] = y.reshape(blk, t, v)


def kernel(input_ids, embed_table, head_w):
    b, t = input_ids.shape
    v, d = embed_table.shape
    ids = input_ids.astype(jnp.int32).reshape(b * t, 1)
    embed_table = embed_table.astype(jnp.bfloat16)
    head_w = head_w.astype(jnp.bfloat16)
    grid = (b // _BLK,)
    return pl.pallas_call(
        _body,
        grid=grid,
        in_specs=[
            pl.BlockSpec((_BLK * t, 1), lambda i: (i, 0)),
            pl.BlockSpec((v, d), lambda i: (0, 0)),
            pl.BlockSpec((v, d), lambda i: (0, 0)),
        ],
        out_specs=pl.BlockSpec((_BLK, t, v), lambda i: (i, 0, 0)),
        out_shape=jax.ShapeDtypeStruct((b, t, v), jnp.float32),
    )(ids, embed_table, head_w)
